# baseline (speedup 1.0000x reference)
import functools

import jax
import jax.numpy as jnp
from jax import lax
from jax.experimental import pallas as pl
from jax.experimental.pallas import tpu as pltpu
from jax.experimental.pallas import tpu_sc as plsc

TOTAL_DIM = 366
TAB_DIM = 128
N_INV = 87

NC = 2
NS = 16
NW = NC * NS

CHUNK = 128


def _body(table_hbm, idx_hbm, zeros_hbm, out_hbm,
          idx0, idx1, blk0, blk1, gsem0, gsem1, wsem0, wsem1):
    wid = lax.axis_index("s") * NC + lax.axis_index("c")
    n = out_hbm.shape[0]
    rows_per_w = n // NW
    pairs = rows_per_w // (2 * CHUNK)
    base = wid * rows_per_w

    pltpu.sync_copy(zeros_hbm, blk0)
    pltpu.sync_copy(zeros_hbm, blk1)

    bufs = ((idx0, blk0, gsem0, wsem0), (idx1, blk1, gsem1, wsem1))

    def do_chunk(b, idx_v, blk, gsem, wsem, first):
        @pl.when(jnp.logical_not(first))
        def _():
            pltpu.make_async_copy(blk, out_hbm.at[pl.ds(b, CHUNK)], wsem).wait()
        pltpu.sync_copy(idx_hbm.at[pl.ds(b, CHUNK)], idx_v)
        pltpu.async_copy(
            table_hbm.at[idx_v], blk.at[:, pl.ds(0, TAB_DIM)], gsem
        ).wait()
        pltpu.async_copy(blk, out_hbm.at[pl.ds(b, CHUNK)], wsem)

    def pair_step(p, carry):
        first = p == 0
        for k, (idx_v, blk, gsem, wsem) in enumerate(bufs):
            do_chunk(base + (2 * p + k) * CHUNK, idx_v, blk, gsem, wsem, first)
        return carry

    lax.fori_loop(0, pairs, pair_step, 0)

    for _, blk, _, wsem in bufs:
        pltpu.make_async_copy(blk, out_hbm.at[pl.ds(base, CHUNK)], wsem).wait()


@jax.jit
def kernel(Z, emb_table):
    n = Z.shape[0]
    padded = jnp.zeros((N_INV, TAB_DIM), jnp.float32).at[:, :N_INV].set(emb_table)
    idx = Z.reshape(n)
    zeros_blk = jnp.zeros((CHUNK, TOTAL_DIM), jnp.float32)

    mesh = plsc.VectorSubcoreMesh(core_axis_name="c", subcore_axis_name="s")
    run = pl.kernel(
        _body,
        out_type=jax.ShapeDtypeStruct((n, TOTAL_DIM), jnp.float32),
        mesh=mesh,
        scratch_types=[
            pltpu.VMEM((CHUNK,), jnp.int32),
            pltpu.VMEM((CHUNK,), jnp.int32),
            pltpu.VMEM((CHUNK, TOTAL_DIM), jnp.float32),
            pltpu.VMEM((CHUNK, TOTAL_DIM), jnp.float32),
            pltpu.SemaphoreType.DMA,
            pltpu.SemaphoreType.DMA,
            pltpu.SemaphoreType.DMA,
            pltpu.SemaphoreType.DMA,
        ],
    )
    return run(padded, idx, zeros_blk)


# speedup vs baseline: 3.7418x; 3.7418x over previous
import functools

import jax
import jax.numpy as jnp
from jax import lax
from jax.experimental import pallas as pl
from jax.experimental.pallas import tpu as pltpu
from jax.experimental.pallas import tpu_sc as plsc

TOTAL_DIM = 366
N_INV = 87
TAB_ROWS = 88
TAB_COLS = 128

NC = 2
NS = 16
NW = NC * NS

CHUNK = 128
LANES = 16
GROUPS = CHUNK // LANES
C_UNROLL = 4


def _body(tab_hbm, idx_hbm, zeros_hbm, out_hbm,
          tab_v, idx0, idx1, blk0, blk1, wsem0, wsem1):
    wid = lax.axis_index("s") * NC + lax.axis_index("c")
    n = out_hbm.shape[1]
    cols_per_w = n // NW
    pairs = cols_per_w // (2 * CHUNK)
    base = wid * cols_per_w

    pltpu.sync_copy(tab_hbm, tab_v)
    pltpu.sync_copy(zeros_hbm, blk0)
    pltpu.sync_copy(zeros_hbm, blk1)

    bufs = ((idx0, blk0, wsem0), (idx1, blk1, wsem1))

    def do_block(b, idx_v, blk, wsem, first):
        @pl.when(jnp.logical_not(first))
        def _():
            pltpu.make_async_copy(blk, out_hbm.at[:, pl.ds(b, CHUNK)], wsem).wait()
        pltpu.sync_copy(idx_hbm.at[pl.ds(b, CHUNK)], idx_v)
        zs = [idx_v[pl.ds(g * LANES, LANES)] for g in range(GROUPS)]

        def fill(i, carry):
            for u in range(C_UNROLL):
                c = i * C_UNROLL + u
                c_vec = jnp.full((LANES,), 0, jnp.int32) + c
                for g in range(GROUPS):
                    blk[c, pl.ds(g * LANES, LANES)] = plsc.load_gather(
                        tab_v, [c_vec, zs[g]]
                    )
            return carry

        lax.fori_loop(0, TAB_ROWS // C_UNROLL, fill, 0)
        pltpu.async_copy(blk, out_hbm.at[:, pl.ds(b, CHUNK)], wsem)

    def pair_step(p, carry):
        first = p == 0
        for k, (idx_v, blk, wsem) in enumerate(bufs):
            do_block(base + (2 * p + k) * CHUNK, idx_v, blk, wsem, first)
        return carry

    lax.fori_loop(0, pairs, pair_step, 0)

    for _, blk, wsem in bufs:
        pltpu.make_async_copy(blk, out_hbm.at[:, pl.ds(base, CHUNK)], wsem).wait()


@jax.jit
def kernel(Z, emb_table):
    n = Z.shape[0]
    tab_t = (
        jnp.zeros((TAB_ROWS, TAB_COLS), jnp.float32)
        .at[:N_INV, :N_INV]
        .set(emb_table.T)
    )
    idx = Z.reshape(n)
    zeros_blk = jnp.zeros((TOTAL_DIM, CHUNK), jnp.float32)

    mesh = plsc.VectorSubcoreMesh(core_axis_name="c", subcore_axis_name="s")
    run = pl.kernel(
        _body,
        out_type=jax.ShapeDtypeStruct((TOTAL_DIM, n), jnp.float32),
        mesh=mesh,
        scratch_types=[
            pltpu.VMEM((TAB_ROWS, TAB_COLS), jnp.float32),
            pltpu.VMEM((CHUNK,), jnp.int32),
            pltpu.VMEM((CHUNK,), jnp.int32),
            pltpu.VMEM((TOTAL_DIM, CHUNK), jnp.float32),
            pltpu.VMEM((TOTAL_DIM, CHUNK), jnp.float32),
            pltpu.SemaphoreType.DMA,
            pltpu.SemaphoreType.DMA,
        ],
        compiler_params=pltpu.CompilerParams(needs_layout_passes=False),
    )
    out_t = run(tab_t, idx, zeros_blk)
    return out_t.T


# speedup vs baseline: 4.3439x; 1.1609x over previous
import functools

import jax
import jax.numpy as jnp
from jax import lax
from jax.experimental import pallas as pl
from jax.experimental.pallas import tpu as pltpu
from jax.experimental.pallas import tpu_sc as plsc

TOTAL_DIM = 366
N_INV = 87
TAB_ROWS = 88
TAB_COLS = 128

NC = 2
NS = 16
NW = NC * NS

CHUNK = 128
LANES = 16
GROUPS = CHUNK // LANES
C_UNROLL = 4


def _body(tab_hbm, idx_hbm, zeros_hbm, out_hbm,
          tab_v, idx0, idx1, blk0, blk1, wsem0, wsem1, isem0, isem1):
    wid = lax.axis_index("s") * NC + lax.axis_index("c")
    n = out_hbm.shape[1]
    cols_per_w = n // NW
    pairs = cols_per_w // (2 * CHUNK)
    base = wid * cols_per_w

    pltpu.sync_copy(tab_hbm, tab_v)
    pltpu.sync_copy(zeros_hbm, blk0)
    pltpu.sync_copy(zeros_hbm, blk1)

    bufs = ((idx0, blk0, wsem0, isem0), (idx1, blk1, wsem1, isem1))

    pltpu.async_copy(idx_hbm.at[pl.ds(base, CHUNK)], idx0, isem0)

    def do_block(b, idx_v, blk, wsem, isem, nxt, first):
        @pl.when(jnp.logical_not(first))
        def _():
            pltpu.make_async_copy(blk, out_hbm.at[:, pl.ds(b, CHUNK)], wsem).wait()
        pltpu.make_async_copy(idx_hbm.at[pl.ds(b, CHUNK)], idx_v, isem).wait()
        nxt_idx, nxt_isem = nxt
        b_next = lax.rem(b + CHUNK, n)
        pltpu.async_copy(idx_hbm.at[pl.ds(b_next, CHUNK)], nxt_idx, nxt_isem)
        zs = [idx_v[pl.ds(g * LANES, LANES)] for g in range(GROUPS)]

        def fill(i, carry):
            for u in range(C_UNROLL):
                c = i * C_UNROLL + u
                c_vec = jnp.full((LANES,), 0, jnp.int32) + c
                for g in range(GROUPS):
                    blk[c, pl.ds(g * LANES, LANES)] = plsc.load_gather(
                        tab_v, [c_vec, zs[g]]
                    )
            return carry

        lax.fori_loop(0, TAB_ROWS // C_UNROLL, fill, 0)
        pltpu.async_copy(blk, out_hbm.at[:, pl.ds(b, CHUNK)], wsem)

    def pair_step(p, carry):
        first = p == 0
        for k, (idx_v, blk, wsem, isem) in enumerate(bufs):
            other = bufs[1 - k]
            do_block(base + (2 * p + k) * CHUNK, idx_v, blk, wsem, isem,
                     (other[0], other[3]), first)
        return carry

    lax.fori_loop(0, pairs, pair_step, 0)

    for idx_v, blk, wsem, _ in bufs:
        pltpu.make_async_copy(blk, out_hbm.at[:, pl.ds(base, CHUNK)], wsem).wait()
    pltpu.make_async_copy(idx_hbm.at[pl.ds(base, CHUNK)], idx0, isem0).wait()


@jax.jit
def kernel(Z, emb_table):
    n = Z.shape[0]
    tab_t = (
        jnp.zeros((TAB_ROWS, TAB_COLS), jnp.float32)
        .at[:N_INV, :N_INV]
        .set(emb_table.T)
    )
    idx = Z.reshape(n)
    zeros_blk = jnp.zeros((TOTAL_DIM, CHUNK), jnp.float32)

    mesh = plsc.VectorSubcoreMesh(core_axis_name="c", subcore_axis_name="s")
    run = pl.kernel(
        _body,
        out_type=jax.ShapeDtypeStruct((TOTAL_DIM, n), jnp.float32),
        mesh=mesh,
        scratch_types=[
            pltpu.VMEM((TAB_ROWS, TAB_COLS), jnp.float32),
            pltpu.VMEM((CHUNK,), jnp.int32),
            pltpu.VMEM((CHUNK,), jnp.int32),
            pltpu.VMEM((TOTAL_DIM, CHUNK), jnp.float32),
            pltpu.VMEM((TOTAL_DIM, CHUNK), jnp.float32),
            pltpu.SemaphoreType.DMA,
            pltpu.SemaphoreType.DMA,
            pltpu.SemaphoreType.DMA,
            pltpu.SemaphoreType.DMA,
        ],
        compiler_params=pltpu.CompilerParams(needs_layout_passes=False),
    )
    out_t = run(tab_t, idx, zeros_blk)
    return out_t.T
